# baseline (speedup 1.0000x reference)
import jax
import jax.numpy as jnp
from jax import lax
from jax.experimental import pallas as pl
from jax.experimental.pallas import tpu as pltpu
from jax.experimental.pallas import tpu_sc as plsc

ALPHA = 0.4
BETA = 0.3
C0 = 1.0 - ALPHA - BETA
V = 1000
S = 2048
B = 4
T_HASH = 8192
N = S * B
NW = 32
PER_W = N // NW
CB = 16
NCH = PER_W // CB
ROW = V
NSL = ROW // 16
TAIL = ROW - 16

_LOG_K = 8.262958405176314e-08
_Q0 = -89.0671764482819
_Q1 = 2.099108045049971
_Q2 = -1.4424810126299674
_Q3 = 0.4358618497882933
_Q4 = -0.05486285286409639


def _fastlog(x):
    b = plsc.bitcast(x, jnp.int32)
    zf = b.astype(jnp.float32)
    m = plsc.bitcast((b & 0x007FFFFF) | 0x3F800000, jnp.float32)
    r = _Q4 * m + _Q3
    r = r * m + _Q2
    r = r * m + _Q1
    r = r * m + _Q0
    return zf * _LOG_K + r


def _sc_body(text_h, uni_h, big_h, tri_h, out_h,
             txt_v, curi_v, trii_v, unis_v, big_v, tri_v, out_v, sem):
    cid = lax.axis_index("c")
    sid = lax.axis_index("s")
    wid = sid * 2 + cid
    base = wid * PER_W
    lanes = lax.broadcasted_iota(jnp.int32, (16,), 0)

    pltpu.sync_copy(text_h, txt_v)
    pltpu.sync_copy(uni_h, unis_v)

    def scale_uni(j, _):
        off = j * 16
        unis_v[pl.ds(off, 16)] = C0 * unis_v[pl.ds(off, 16)]
        return 0
    lax.fori_loop(0, NSL, scale_uni, 0)
    tail_u = unis_v[pl.ds(TAIL, 16)]
    unis_v[pl.ds(TAIL, 16)] = jnp.where(lanes >= 8, C0 * tail_u, tail_u)

    def idx_body(s, _):
        cur = txt_v[pl.ds(base + s * 16 + 8, 16)]
        prev = txt_v[pl.ds(base + s * 16 + 4, 16)]
        tri = (prev * V + cur) & (T_HASH - 1)
        curi_v[s, :] = cur
        trii_v[s, :] = tri
        return 0
    lax.fori_loop(0, PER_W // 16, idx_body, 0)

    def chunk_body(c, _):
        pltpu.async_copy(big_h.at[curi_v.at[c]], big_v, sem).wait()
        pltpu.async_copy(tri_h.at[trii_v.at[c]], tri_v, sem).wait()

        def row_body(r, _):
            k = base + c * CB + r
            betak = jnp.where(jnp.broadcast_to(k, (16,)) >= 2 * B,
                              jnp.float32(BETA), jnp.float32(0.0))

            def p1(j, acc):
                off = j * 16
                p = (unis_v[pl.ds(off, 16)]
                     + ALPHA * big_v[r, pl.ds(off, 16)]
                     + betak * tri_v[r, pl.ds(off, 16)])
                out_v[r, pl.ds(off, 16)] = _fastlog(p)
                return acc + p
            acc = lax.fori_loop(0, NSL, p1, jnp.zeros((16,), jnp.float32))

            p = (unis_v[pl.ds(TAIL, 16)]
                 + ALPHA * big_v[r, pl.ds(TAIL, 16)]
                 + betak * tri_v[r, pl.ds(TAIL, 16)])
            out_v[r, pl.ds(TAIL, 16)] = _fastlog(p)
            acc = acc + jnp.where(lanes >= 8, p, 0.0)

            s_tot = jnp.sum(acc) + 1e-10
            lsv = _fastlog(jnp.broadcast_to(s_tot, (16,)))

            def p2(j, _):
                off = j * 16
                out_v[r, pl.ds(off, 16)] = out_v[r, pl.ds(off, 16)] - lsv
                return 0
            lax.fori_loop(0, NSL, p2, 0)
            tail_sub = jnp.where(lanes >= 8, lsv, jnp.float32(0.0))
            out_v[r, pl.ds(TAIL, 16)] = out_v[r, pl.ds(TAIL, 16)] - tail_sub
            return 0
        lax.fori_loop(0, CB, row_body, 0)

        pltpu.sync_copy(out_v, out_h.at[pl.ds(base + c * CB, CB)])
        return 0
    lax.fori_loop(0, NCH, chunk_body, 0)


@jax.jit
def kernel(text, unigram, bigram_table, trigram_table):
    textf = jnp.pad(text.reshape(N), (8, 0))
    mesh = plsc.VectorSubcoreMesh(core_axis_name="c", subcore_axis_name="s")
    out = pl.kernel(
        _sc_body,
        out_type=jax.ShapeDtypeStruct((N, ROW), jnp.float32),
        mesh=mesh,
        compiler_params=pltpu.CompilerParams(
            needs_layout_passes=False, use_tc_tiling_on_sc=False),
        scratch_types=[
            pltpu.VMEM((N + 8,), jnp.int32),
            pltpu.VMEM((NCH, CB), jnp.int32),
            pltpu.VMEM((NCH, CB), jnp.int32),
            pltpu.VMEM((ROW,), jnp.float32),
            pltpu.VMEM((CB, ROW), jnp.float32),
            pltpu.VMEM((CB, ROW), jnp.float32),
            pltpu.VMEM((CB, ROW), jnp.float32),
            pltpu.SemaphoreType.DMA,
        ],
    )(textf, unigram, bigram_table, trigram_table)
    return out.reshape(S, B, V)


# speedup vs baseline: 2.0215x; 2.0215x over previous
import jax
import jax.numpy as jnp
from jax import lax
from jax.experimental import pallas as pl
from jax.experimental.pallas import tpu as pltpu
from jax.experimental.pallas import tpu_sc as plsc

ALPHA = 0.4
BETA = 0.3
C0 = 1.0 - ALPHA - BETA
V = 1000
S = 2048
B = 4
T_HASH = 8192
N = S * B
NW = 32
PER_W = N // NW
CB = 16
NCH = PER_W // CB
ROW = V
NSL = ROW // 16
TAIL = ROW - 16

_LOG_K = 8.262958405176314e-08
_Q0 = -89.0671764482819
_Q1 = 2.099108045049971
_Q2 = -1.4424810126299674
_Q3 = 0.4358618497882933
_Q4 = -0.05486285286409639


def _fastlog(x):
    b = plsc.bitcast(x, jnp.int32)
    zf = b.astype(jnp.float32)
    m = plsc.bitcast((b & 0x007FFFFF) | 0x3F800000, jnp.float32)
    r = _Q4 * m + _Q3
    r = r * m + _Q2
    r = r * m + _Q1
    r = r * m + _Q0
    return zf * _LOG_K + r


def _sc_body(text_h, uni_h, big_h, tri_h, out_h,
             txt_v, curi_v, trii_v, unis_v, big_v, tri_v, out_v, sem):
    cid = lax.axis_index("c")
    sid = lax.axis_index("s")
    wid = sid * 2 + cid
    base = wid * PER_W
    lanes = lax.broadcasted_iota(jnp.int32, (16,), 0)

    pltpu.sync_copy(text_h, txt_v)
    pltpu.sync_copy(uni_h, unis_v)

    def scale_uni(j, _):
        off = j * 16
        unis_v[pl.ds(off, 16)] = C0 * unis_v[pl.ds(off, 16)]
        return 0
    lax.fori_loop(0, NSL, scale_uni, 0)
    tail_u = unis_v[pl.ds(TAIL, 16)]
    unis_v[pl.ds(TAIL, 16)] = jnp.where(lanes >= 8, C0 * tail_u, tail_u)

    def idx_body(s, _):
        cur = txt_v[pl.ds(base + s * 16 + 8, 16)]
        prev = txt_v[pl.ds(base + s * 16 + 4, 16)]
        tri = (prev * V + cur) & (T_HASH - 1)
        curi_v[s, :] = cur
        trii_v[s, :] = tri
        return 0
    lax.fori_loop(0, PER_W // 16, idx_body, 0)

    def chunk_body(c, _):
        pltpu.async_copy(big_h.at[curi_v.at[c]], big_v, sem).wait()
        pltpu.async_copy(tri_h.at[trii_v.at[c]], tri_v, sem).wait()

        def row_body(r, _):
            k = base + c * CB + r
            betak = jnp.where(jnp.broadcast_to(k, (16,)) >= 2 * B,
                              jnp.float32(BETA), jnp.float32(0.0))

            zero16 = jnp.zeros((16,), jnp.float32)

            @plsc.parallel_loop(0, NSL, unroll=4,
                                carry=(zero16, zero16, zero16, zero16))
            def p1(j, accs):
                a0, a1, a2, a3 = accs
                off = j * 16
                p = (unis_v[pl.ds(off, 16)]
                     + ALPHA * big_v[r, pl.ds(off, 16)]
                     + betak * tri_v[r, pl.ds(off, 16)])
                out_v[r, pl.ds(off, 16)] = _fastlog(p)
                return (a1, a2, a3, a0 + p)
            a0, a1, a2, a3 = p1
            acc = (a0 + a1) + (a2 + a3)

            p = (unis_v[pl.ds(TAIL, 16)]
                 + ALPHA * big_v[r, pl.ds(TAIL, 16)]
                 + betak * tri_v[r, pl.ds(TAIL, 16)])
            out_v[r, pl.ds(TAIL, 16)] = _fastlog(p)
            acc = acc + jnp.where(lanes >= 8, p, 0.0)

            s_tot = jnp.sum(acc) + 1e-10
            lsv = _fastlog(jnp.broadcast_to(s_tot, (16,)))

            @plsc.parallel_loop(0, NSL, unroll=4)
            def p2(j):
                off = j * 16
                out_v[r, pl.ds(off, 16)] = out_v[r, pl.ds(off, 16)] - lsv
            tail_sub = jnp.where(lanes >= 8, lsv, jnp.float32(0.0))
            out_v[r, pl.ds(TAIL, 16)] = out_v[r, pl.ds(TAIL, 16)] - tail_sub
            return 0
        lax.fori_loop(0, CB, row_body, 0)

        pltpu.sync_copy(out_v, out_h.at[pl.ds(base + c * CB, CB)])
        return 0
    lax.fori_loop(0, NCH, chunk_body, 0)


@jax.jit
def kernel(text, unigram, bigram_table, trigram_table):
    textf = jnp.pad(text.reshape(N), (8, 0))
    mesh = plsc.VectorSubcoreMesh(core_axis_name="c", subcore_axis_name="s")
    out = pl.kernel(
        _sc_body,
        out_type=jax.ShapeDtypeStruct((N, ROW), jnp.float32),
        mesh=mesh,
        compiler_params=pltpu.CompilerParams(
            needs_layout_passes=False, use_tc_tiling_on_sc=False),
        scratch_types=[
            pltpu.VMEM((N + 8,), jnp.int32),
            pltpu.VMEM((NCH, CB), jnp.int32),
            pltpu.VMEM((NCH, CB), jnp.int32),
            pltpu.VMEM((ROW,), jnp.float32),
            pltpu.VMEM((CB, ROW), jnp.float32),
            pltpu.VMEM((CB, ROW), jnp.float32),
            pltpu.VMEM((CB, ROW), jnp.float32),
            pltpu.SemaphoreType.DMA,
        ],
    )(textf, unigram, bigram_table, trigram_table)
    return out.reshape(S, B, V)


# speedup vs baseline: 2.3724x; 1.1736x over previous
import jax
import jax.numpy as jnp
from jax import lax
from jax.experimental import pallas as pl
from jax.experimental.pallas import tpu as pltpu
from jax.experimental.pallas import tpu_sc as plsc

ALPHA = 0.4
BETA = 0.3
C0 = 1.0 - ALPHA - BETA
V = 1000
S = 2048
B = 4
T_HASH = 8192
N = S * B
NW = 32
PER_W = N // NW
CB = 16
NCH = PER_W // CB
ROW = V
NSL = ROW // 16
TAIL = ROW - 16

_LOG_K = 8.262958405176314e-08
_Q0 = -89.0671764482819
_Q1 = 2.099108045049971
_Q2 = -1.4424810126299674
_Q3 = 0.4358618497882933
_Q4 = -0.05486285286409639


def _fastlog(x):
    b = plsc.bitcast(x, jnp.int32)
    zf = b.astype(jnp.float32)
    m = plsc.bitcast((b & 0x007FFFFF) | 0x3F800000, jnp.float32)
    r = _Q4 * m + _Q3
    r = r * m + _Q2
    r = r * m + _Q1
    r = r * m + _Q0
    return zf * _LOG_K + r


def _sc_body(text_h, uni_h, big_h, tri_h, out_h,
             txt_v, curi_v, trii_v, unis_v, big_v, tri_v, out_v,
             sem_g0, sem_g1, sem_o0, sem_o1):
    sem_g = (sem_g0, sem_g1)
    sem_o = (sem_o0, sem_o1)
    cid = lax.axis_index("c")
    sid = lax.axis_index("s")
    wid = sid * 2 + cid
    base = wid * PER_W
    lanes = lax.broadcasted_iota(jnp.int32, (16,), 0)

    pltpu.sync_copy(text_h, txt_v)
    pltpu.sync_copy(uni_h, unis_v)

    def scale_uni(j, _):
        off = j * 16
        unis_v[pl.ds(off, 16)] = C0 * unis_v[pl.ds(off, 16)]
        return 0
    lax.fori_loop(0, NSL, scale_uni, 0)
    tail_u = unis_v[pl.ds(TAIL, 16)]
    unis_v[pl.ds(TAIL, 16)] = jnp.where(lanes >= 8, C0 * tail_u, tail_u)

    def idx_body(s, _):
        cur = txt_v[pl.ds(base + s * 16 + 8, 16)]
        prev = txt_v[pl.ds(base + s * 16 + 4, 16)]
        tri = (prev * V + cur) & (T_HASH - 1)
        curi_v[s, :] = cur
        trii_v[s, :] = tri
        return 0
    lax.fori_loop(0, PER_W // 16, idx_body, 0)

    def gathers(c, buf):
        cb = pltpu.make_async_copy(big_h.at[curi_v.at[c]], big_v.at[buf],
                                   sem_g[buf])
        ct = pltpu.make_async_copy(tri_h.at[trii_v.at[c]], tri_v.at[buf],
                                   sem_g[buf])
        return cb, ct

    def out_copy(c, buf):
        return pltpu.make_async_copy(out_v.at[buf],
                                     out_h.at[pl.ds(base + c * CB, CB)],
                                     sem_o[buf])

    def chunk_body(c, buf):
        bv = big_v.at[buf]
        tv = tri_v.at[buf]
        ov = out_v.at[buf]

        def row_body(r, _):
            k = base + c * CB + r
            betak = jnp.where(jnp.broadcast_to(k, (16,)) >= 2 * B,
                              jnp.float32(BETA), jnp.float32(0.0))

            zero16 = jnp.zeros((16,), jnp.float32)

            @plsc.parallel_loop(0, NSL, unroll=4,
                                carry=(zero16, zero16, zero16, zero16))
            def p1(j, accs):
                a0, a1, a2, a3 = accs
                off = j * 16
                p = (unis_v[pl.ds(off, 16)]
                     + ALPHA * bv[r, pl.ds(off, 16)]
                     + betak * tv[r, pl.ds(off, 16)])
                ov[r, pl.ds(off, 16)] = _fastlog(p)
                return (a1, a2, a3, a0 + p)
            a0, a1, a2, a3 = p1
            acc = (a0 + a1) + (a2 + a3)

            p = (unis_v[pl.ds(TAIL, 16)]
                 + ALPHA * bv[r, pl.ds(TAIL, 16)]
                 + betak * tv[r, pl.ds(TAIL, 16)])
            ov[r, pl.ds(TAIL, 16)] = _fastlog(p)
            acc = acc + jnp.where(lanes >= 8, p, 0.0)

            s_tot = jnp.sum(acc) + 1e-10
            lsv = _fastlog(jnp.broadcast_to(s_tot, (16,)))

            @plsc.parallel_loop(0, NSL, unroll=4)
            def p2(j):
                off = j * 16
                ov[r, pl.ds(off, 16)] = ov[r, pl.ds(off, 16)] - lsv
            tail_sub = jnp.where(lanes >= 8, lsv, jnp.float32(0.0))
            ov[r, pl.ds(TAIL, 16)] = ov[r, pl.ds(TAIL, 16)] - tail_sub
            return 0
        lax.fori_loop(0, CB, row_body, 0)

    g0b, g0t = gathers(0, 0)
    g0b.start()
    g0t.start()
    for c in range(NCH):
        buf = c % 2
        if c + 1 < NCH:
            nb, nt = gathers(c + 1, 1 - buf)
            nb.start()
            nt.start()
        gb, gt = gathers(c, buf)
        gb.wait()
        gt.wait()
        if c >= 2:
            out_copy(c - 2, buf).wait()
        chunk_body(c, buf)
        out_copy(c, buf).start()
    out_copy(NCH - 2, NCH % 2).wait()
    out_copy(NCH - 1, (NCH - 1) % 2).wait()


@jax.jit
def kernel(text, unigram, bigram_table, trigram_table):
    textf = jnp.pad(text.reshape(N), (8, 0))
    mesh = plsc.VectorSubcoreMesh(core_axis_name="c", subcore_axis_name="s")
    out = pl.kernel(
        _sc_body,
        out_type=jax.ShapeDtypeStruct((N, ROW), jnp.float32),
        mesh=mesh,
        compiler_params=pltpu.CompilerParams(
            needs_layout_passes=False, use_tc_tiling_on_sc=False),
        scratch_types=[
            pltpu.VMEM((N + 8,), jnp.int32),
            pltpu.VMEM((NCH, CB), jnp.int32),
            pltpu.VMEM((NCH, CB), jnp.int32),
            pltpu.VMEM((ROW,), jnp.float32),
            pltpu.VMEM((2, CB, ROW), jnp.float32),
            pltpu.VMEM((2, CB, ROW), jnp.float32),
            pltpu.VMEM((2, CB, ROW), jnp.float32),
            pltpu.SemaphoreType.DMA,
            pltpu.SemaphoreType.DMA,
            pltpu.SemaphoreType.DMA,
            pltpu.SemaphoreType.DMA,
        ],
    )(textf, unigram, bigram_table, trigram_table)
    return out.reshape(S, B, V)


# speedup vs baseline: 2.5592x; 1.0788x over previous
import jax
import jax.numpy as jnp
from jax import lax
from jax.experimental import pallas as pl
from jax.experimental.pallas import tpu as pltpu
from jax.experimental.pallas import tpu_sc as plsc

ALPHA = 0.4
BETA = 0.3
C0 = 1.0 - ALPHA - BETA
V = 1000
S = 2048
B = 4
T_HASH = 8192
N = S * B
NW = 32
PER_W = N // NW
CB = 16
NCH = PER_W // CB
ROWP = 1024
NSL = 63


def _sc_body(text_h, uni_h, big_h, tri_h, out_h,
             txt_v, curi_v, trii_v, unis_v, big_v, tri_v, out_v,
             sem_g0, sem_g1, sem_o0, sem_o1):
    sem_g = (sem_g0, sem_g1)
    sem_o = (sem_o0, sem_o1)
    cid = lax.axis_index("c")
    sid = lax.axis_index("s")
    wid = sid * 2 + cid
    base = wid * PER_W

    pltpu.sync_copy(text_h, txt_v)
    pltpu.sync_copy(uni_h, unis_v.at[pl.ds(0, V)])

    @plsc.parallel_loop(0, NSL, unroll=4)
    def scale_uni(j):
        off = j * 16
        unis_v[pl.ds(off, 16)] = C0 * unis_v[pl.ds(off, 16)]

    def idx_body(s_, _):
        cur = txt_v[pl.ds(base + s_ * 16 + 8, 16)]
        prev = txt_v[pl.ds(base + s_ * 16 + 4, 16)]
        tri = (prev * V + cur) & (T_HASH - 1)
        curi_v[s_, :] = cur
        trii_v[s_, :] = tri
        return 0
    lax.fori_loop(0, PER_W // 16, idx_body, 0)

    def gathers(c, buf):
        cb = pltpu.make_async_copy(big_h.at[curi_v.at[c]], big_v.at[buf],
                                   sem_g[buf])
        ct = pltpu.make_async_copy(tri_h.at[trii_v.at[c]], tri_v.at[buf],
                                   sem_g[buf])
        return cb, ct

    def out_copy(c, buf):
        return pltpu.make_async_copy(out_v.at[buf],
                                     out_h.at[pl.ds(base + c * CB, CB)],
                                     sem_o[buf])

    def chunk_body(c, buf):
        bv = big_v.at[buf]
        tv = tri_v.at[buf]
        ov = out_v.at[buf]

        def row_body(r, _):
            k = base + c * CB + r
            betak = jnp.where(jnp.broadcast_to(k, (16,)) >= 2 * B,
                              jnp.float32(BETA), jnp.float32(0.0))

            @plsc.parallel_loop(0, NSL, unroll=4)
            def p1(j):
                ct_ = j // 8
                cl = (j % 8) * 16
                off = j * 16
                p = (unis_v[pl.ds(off, 16)]
                     + ALPHA * bv[r, ct_, pl.ds(cl, 16)]
                     + betak * tv[r, ct_, pl.ds(cl, 16)])
                ov[r, ct_, pl.ds(cl, 16)] = p
            return 0
        lax.fori_loop(0, CB, row_body, 0)

    g0b, g0t = gathers(0, 0)
    g0b.start()
    g0t.start()
    for c in range(NCH):
        buf = c % 2
        if c + 1 < NCH:
            nb, nt = gathers(c + 1, 1 - buf)
            nb.start()
            nt.start()
        gb, gt = gathers(c, buf)
        gb.wait()
        gt.wait()
        if c >= 2:
            out_copy(c - 2, buf).wait()
        chunk_body(c, buf)
        out_copy(c, buf).start()
    out_copy(NCH - 2, NCH % 2).wait()
    out_copy(NCH - 1, (NCH - 1) % 2).wait()


def _tc_epilogue(p_ref, o_ref):
    x = p_ref[...]
    ct_ = lax.broadcasted_iota(jnp.int32, x.shape, 1)
    cl = lax.broadcasted_iota(jnp.int32, x.shape, 2)
    valid = (ct_ * 128 + cl) < V
    xz = jnp.where(valid, x, 0.0)
    s = jnp.sum(xz, axis=(1, 2), keepdims=True) + 1e-10
    q = jnp.log(1e-10 + xz / s)
    y = q.reshape(64, ROWP)[:, :V].reshape(16, B, V)
    o_ref[...] = y


@jax.jit
def kernel(text, unigram, bigram_table, trigram_table):
    textf = jnp.pad(text.reshape(N), (8, 0))
    big3 = jnp.pad(bigram_table, ((0, 0), (0, 24))).reshape(V, 8, 128)
    tri3 = jnp.pad(trigram_table, ((0, 0), (0, 24))).reshape(T_HASH, 8, 128)

    mesh = plsc.VectorSubcoreMesh(core_axis_name="c", subcore_axis_name="s")
    p3 = pl.kernel(
        _sc_body,
        out_type=jax.ShapeDtypeStruct((N, 8, 128), jnp.float32),
        mesh=mesh,
        compiler_params=pltpu.CompilerParams(
            needs_layout_passes=False, use_tc_tiling_on_sc=False),
        scratch_types=[
            pltpu.VMEM((N + 8,), jnp.int32),
            pltpu.VMEM((NCH, CB), jnp.int32),
            pltpu.VMEM((NCH, CB), jnp.int32),
            pltpu.VMEM((ROWP,), jnp.float32),
            pltpu.VMEM((2, CB, 8, 128), jnp.float32),
            pltpu.VMEM((2, CB, 8, 128), jnp.float32),
            pltpu.VMEM((2, CB, 8, 128), jnp.float32),
            pltpu.SemaphoreType.DMA,
            pltpu.SemaphoreType.DMA,
            pltpu.SemaphoreType.DMA,
            pltpu.SemaphoreType.DMA,
        ],
    )(textf, unigram, big3, tri3)

    out = pl.pallas_call(
        _tc_epilogue,
        grid=(N // 64,),
        in_specs=[pl.BlockSpec((64, 8, 128), lambda i: (i, 0, 0))],
        out_specs=pl.BlockSpec((16, B, V), lambda i: (i, 0, 0)),
        out_shape=jax.ShapeDtypeStruct((S, B, V), jnp.float32),
    )(p3)
    return out
